# Initial kernel scaffold; baseline (speedup 1.0000x reference)
import jax
import jax.numpy as jnp
from jax.experimental import pallas as pl


def kernel(inputs, W1, b1, W2, b2):
    raise NotImplementedError("write your pallas kernel here")


# speedup vs baseline: 1.1630x; 1.1630x over previous
import functools

import jax
import jax.numpy as jnp
from jax.experimental import pallas as pl
from jax.experimental.pallas import tpu as pltpu

_TOPK = 8


def _score_body(x_ref, w1_ref, b1_ref, w2_ref, b2_ref, out_ref, h_acc,
                *, nsteps_k):
    n = pl.program_id(1)
    k = pl.program_id(2)

    @pl.when(k == 0)
    def _():
        h_acc[...] = jnp.zeros_like(h_acc)

    h_acc[...] += jnp.dot(x_ref[...], w1_ref[...],
                          preferred_element_type=jnp.float32)

    @pl.when(k == nsteps_k - 1)
    def _():
        h = jnp.maximum(h_acc[...] + b1_ref[...], 0.0)
        contrib = jnp.dot(h, w2_ref[...], preferred_element_type=jnp.float32)

        @pl.when(n == 0)
        def _():
            out_ref[...] = contrib + b2_ref[...]

        @pl.when(n > 0)
        def _():
            out_ref[...] += contrib


def _route_body(s_ref, router_ref, idx_ref):
    s = s_ref[...]
    num_e = s.shape[-1]
    col = jax.lax.broadcasted_iota(jnp.int32, s.shape, 1)
    neg = jnp.float32(-jnp.inf)
    work = s
    m0 = None
    idxs = []
    for j in range(_TOPK):
        mx = jnp.max(work, axis=1, keepdims=True)
        if j == 0:
            m0 = mx
        amx = jnp.min(jnp.where(work == mx, col, num_e), axis=1,
                      keepdims=True)
        idxs.append(amx)
        work = jnp.where(col == amx, neg, work)
    sel = work == neg
    p = jnp.where(sel, jnp.exp(s - m0), 0.0)
    router_ref[...] = p / jnp.sum(p, axis=1, keepdims=True)
    idx_ref[...] = jnp.concatenate(idxs, axis=1)


def kernel(inputs, W1, b1, W2, b2):
    m, k_dim = inputs.shape
    n_dim = W1.shape[1]
    num_e = W2.shape[1]
    bm, bn, bk = min(512, m), min(1024, n_dim), min(1024, k_dim)
    grid = (m // bm, n_dim // bn, k_dim // bk)

    score = pl.pallas_call(
        functools.partial(_score_body, nsteps_k=grid[2]),
        grid=grid,
        in_specs=[
            pl.BlockSpec((bm, bk), lambda i, j, k: (i, k)),
            pl.BlockSpec((bk, bn), lambda i, j, k: (k, j)),
            pl.BlockSpec((1, bn), lambda i, j, k: (0, j)),
            pl.BlockSpec((bn, num_e), lambda i, j, k: (j, 0)),
            pl.BlockSpec((1, num_e), lambda i, j, k: (0, 0)),
        ],
        out_specs=pl.BlockSpec((bm, num_e), lambda i, j, k: (i, 0)),
        out_shape=jax.ShapeDtypeStruct((m, num_e), jnp.float32),
        scratch_shapes=[pltpu.VMEM((bm, bn), jnp.float32)],
        compiler_params=pltpu.CompilerParams(
            dimension_semantics=("parallel", "arbitrary", "arbitrary")),
    )(inputs, W1, b1.reshape(1, n_dim), W2, b2.reshape(1, num_e))

    bm2 = min(1024, m)
    router, idx = pl.pallas_call(
        _route_body,
        grid=(m // bm2,),
        in_specs=[pl.BlockSpec((bm2, num_e), lambda i: (i, 0))],
        out_specs=[
            pl.BlockSpec((bm2, num_e), lambda i: (i, 0)),
            pl.BlockSpec((bm2, _TOPK), lambda i: (i, 0)),
        ],
        out_shape=[
            jax.ShapeDtypeStruct((m, num_e), jnp.float32),
            jax.ShapeDtypeStruct((m, _TOPK), jnp.int32),
        ],
    )(score)
    return router, idx


# speedup vs baseline: 1.9710x; 1.6948x over previous
import functools

import jax
import jax.numpy as jnp
from jax.experimental import pallas as pl
from jax.experimental.pallas import tpu as pltpu

_TOPK = 8


def _score_body(x_ref, w1_ref, b1_ref, w2_ref, b2_ref, out_ref, h_acc,
                *, nsteps_k):
    n = pl.program_id(1)
    k = pl.program_id(2)

    @pl.when(k == 0)
    def _():
        h_acc[...] = jnp.zeros_like(h_acc)

    h_acc[...] += jnp.dot(x_ref[...], w1_ref[...],
                          preferred_element_type=jnp.float32)

    @pl.when(k == nsteps_k - 1)
    def _():
        h = jnp.maximum(h_acc[...] + b1_ref[...], 0.0)
        contrib = jnp.dot(h, w2_ref[...], preferred_element_type=jnp.float32)

        @pl.when(n == 0)
        def _():
            out_ref[...] = contrib + b2_ref[...]

        @pl.when(n > 0)
        def _():
            out_ref[...] += contrib


def _route_body(s_ref, router_ref, idx_ref):
    s = s_ref[...]
    num_e = s.shape[-1]
    col = jax.lax.broadcasted_iota(jnp.int32, s.shape, 1)
    neg = jnp.float32(-jnp.inf)
    work = s
    m0 = None
    idxs = []
    for j in range(_TOPK):
        mx = jnp.max(work, axis=1, keepdims=True)
        if j == 0:
            m0 = mx
        amx = jnp.min(jnp.where(work == mx, col, num_e), axis=1,
                      keepdims=True)
        idxs.append(amx)
        work = jnp.where(col == amx, neg, work)
    sel = work == neg
    p = jnp.where(sel, jnp.exp(s - m0), 0.0)
    router_ref[...] = p / jnp.sum(p, axis=1, keepdims=True)
    idx_ref[...] = jnp.concatenate(idxs, axis=1)


def kernel(inputs, W1, b1, W2, b2):
    m, k_dim = inputs.shape
    n_dim = W1.shape[1]
    num_e = W2.shape[1]
    bm, bn, bk = min(2048, m), min(2048, n_dim), min(512, k_dim)
    grid = (m // bm, n_dim // bn, k_dim // bk)

    score = pl.pallas_call(
        functools.partial(_score_body, nsteps_k=grid[2]),
        grid=grid,
        in_specs=[
            pl.BlockSpec((bm, bk), lambda i, j, k: (i, k)),
            pl.BlockSpec((bk, bn), lambda i, j, k: (k, j)),
            pl.BlockSpec((1, bn), lambda i, j, k: (0, j)),
            pl.BlockSpec((bn, num_e), lambda i, j, k: (j, 0)),
            pl.BlockSpec((1, num_e), lambda i, j, k: (0, 0)),
        ],
        out_specs=pl.BlockSpec((bm, num_e), lambda i, j, k: (i, 0)),
        out_shape=jax.ShapeDtypeStruct((m, num_e), jnp.float32),
        scratch_shapes=[pltpu.VMEM((bm, bn), jnp.float32)],
        compiler_params=pltpu.CompilerParams(
            dimension_semantics=("parallel", "arbitrary", "arbitrary")),
    )(inputs, W1, b1.reshape(1, n_dim), W2, b2.reshape(1, num_e))

    bm2 = min(1024, m)
    router, idx = pl.pallas_call(
        _route_body,
        grid=(m // bm2,),
        in_specs=[pl.BlockSpec((bm2, num_e), lambda i: (i, 0))],
        out_specs=[
            pl.BlockSpec((bm2, num_e), lambda i: (i, 0)),
            pl.BlockSpec((bm2, _TOPK), lambda i: (i, 0)),
        ],
        out_shape=[
            jax.ShapeDtypeStruct((m, num_e), jnp.float32),
            jax.ShapeDtypeStruct((m, _TOPK), jnp.int32),
        ],
    )(score)
    return router, idx


# speedup vs baseline: 1.9900x; 1.0096x over previous
import functools

import jax
import jax.numpy as jnp
from jax.experimental import pallas as pl
from jax.experimental.pallas import tpu as pltpu

_TOPK = 8


def _score_body(x_ref, w1_ref, b1_ref, w2_ref, b2_ref, out_ref, h_acc,
                *, nsteps_k):
    n = pl.program_id(1)
    k = pl.program_id(2)

    @pl.when(k == 0)
    def _():
        h_acc[...] = jnp.zeros_like(h_acc)

    h_acc[...] += jnp.dot(x_ref[...], w1_ref[...],
                          preferred_element_type=jnp.float32)

    @pl.when(k == nsteps_k - 1)
    def _():
        h = jnp.maximum(h_acc[...] + b1_ref[...], 0.0)
        contrib = jnp.dot(h, w2_ref[...], preferred_element_type=jnp.float32)

        @pl.when(n == 0)
        def _():
            out_ref[...] = contrib + b2_ref[...]

        @pl.when(n > 0)
        def _():
            out_ref[...] += contrib


def _route_body(s_ref, router_ref, idx_ref):
    s = s_ref[...]
    num_e = s.shape[-1]
    col = jax.lax.broadcasted_iota(jnp.int32, s.shape, 1)
    neg = jnp.float32(-jnp.inf)
    work = s
    m0 = None
    idxs = []
    for j in range(_TOPK):
        mx = jnp.max(work, axis=1, keepdims=True)
        if j == 0:
            m0 = mx
        amx = jnp.min(jnp.where(work == mx, col, num_e), axis=1,
                      keepdims=True)
        idxs.append(amx)
        work = jnp.where(col == amx, neg, work)
    sel = work == neg
    p = jnp.where(sel, jnp.exp(s - m0), 0.0)
    router_ref[...] = p / jnp.sum(p, axis=1, keepdims=True)
    idx_ref[...] = jnp.concatenate(idxs, axis=1)


def kernel(inputs, W1, b1, W2, b2):
    m, k_dim = inputs.shape
    n_dim = W1.shape[1]
    num_e = W2.shape[1]
    bm, bn, bk = min(2048, m), min(4096, n_dim), min(256, k_dim)
    grid = (m // bm, n_dim // bn, k_dim // bk)

    score = pl.pallas_call(
        functools.partial(_score_body, nsteps_k=grid[2]),
        grid=grid,
        in_specs=[
            pl.BlockSpec((bm, bk), lambda i, j, k: (i, k)),
            pl.BlockSpec((bk, bn), lambda i, j, k: (k, j)),
            pl.BlockSpec((1, bn), lambda i, j, k: (0, j)),
            pl.BlockSpec((bn, num_e), lambda i, j, k: (j, 0)),
            pl.BlockSpec((1, num_e), lambda i, j, k: (0, 0)),
        ],
        out_specs=pl.BlockSpec((bm, num_e), lambda i, j, k: (i, 0)),
        out_shape=jax.ShapeDtypeStruct((m, num_e), jnp.float32),
        scratch_shapes=[pltpu.VMEM((bm, bn), jnp.float32)],
        compiler_params=pltpu.CompilerParams(
            dimension_semantics=("parallel", "arbitrary", "arbitrary")),
    )(inputs, W1, b1.reshape(1, n_dim), W2, b2.reshape(1, num_e))

    bm2 = min(1024, m)
    router, idx = pl.pallas_call(
        _route_body,
        grid=(m // bm2,),
        in_specs=[pl.BlockSpec((bm2, num_e), lambda i: (i, 0))],
        out_specs=[
            pl.BlockSpec((bm2, num_e), lambda i: (i, 0)),
            pl.BlockSpec((bm2, _TOPK), lambda i: (i, 0)),
        ],
        out_shape=[
            jax.ShapeDtypeStruct((m, num_e), jnp.float32),
            jax.ShapeDtypeStruct((m, _TOPK), jnp.int32),
        ],
    )(score)
    return router, idx
